# baseline (speedup 1.0000x reference)
import jax
import jax.numpy as jnp
from jax import lax
from jax.experimental import pallas as pl
from jax.experimental.pallas import tpu as pltpu
from jax.experimental.pallas import tpu_sc as plsc

C = 768
HW = 112 * 112
TOPK = 384
L = 16
NSUB = 16
CPT = C // NSUB
NVREG = C // L

NBUF = 4
CB = 96
NCHUNK = C // CB


def _mean_body(x_hbm, o_ref, *scratch):
    bufs, sems = scratch[:NBUF], scratch[NBUF:]

    def copy(c):
        b = c % NBUF
        return pltpu.make_async_copy(
            x_hbm.at[pl.ds(c * CB, CB)], bufs[b], sems[b])

    for c in range(min(NBUF, NCHUNK)):
        copy(c).start()
    for c in range(NCHUNK):
        copy(c).wait()
        x = bufs[c % NBUF][...]
        o_ref[pl.ds(c * CB, CB)] = (
            jnp.sum(x, axis=1, keepdims=True) * (1.0 / HW))
        if c + NBUF < NCHUNK:
            copy(c + NBUF).start()


def _channel_means(x2):
    return pl.pallas_call(
        _mean_body,
        in_specs=[pl.BlockSpec(memory_space=pltpu.MemorySpace.HBM)],
        out_specs=pl.BlockSpec(memory_space=pltpu.MemorySpace.VMEM),
        out_shape=jax.ShapeDtypeStruct((C, 1), jnp.float32),
        scratch_shapes=(
            [pltpu.VMEM((CB, HW), jnp.float32)] * NBUF
            + [pltpu.SemaphoreType.DMA] * NBUF),
    )(x2)


def _topk_body(means_hbm, out_hbm, means_v, ranks_mine, ids_mine, out_sh,
               out_v):
    sid = lax.axis_index("s")
    iota = jnp.arange(L, dtype=jnp.int32)
    ones = jnp.ones((L,), jnp.int32)
    zeros = jnp.zeros((L,), jnp.int32)

    pltpu.sync_copy(means_hbm, means_v)

    for tl in range(CPT // L):
        tbase = (sid * (CPT // L) + tl) * L
        v = means_v[pl.ds(tbase, L)]
        idx_t = iota + tbase

        @plsc.parallel_loop(0, NVREG, unroll=4, carry=zeros)
        def acc(m, acc, v=v, idx_t=idx_t):
            u = means_v[pl.ds(m * L, L)]
            for k in range(L):
                us = u.at[jnp.full((L,), k, jnp.int32)].get(
                    mode='promise_in_bounds')
                j = m * L + k
                cond = (us > v) | ((us == v) & (j > idx_t))
                acc = acc + jnp.where(cond, ones, zeros)
            return acc
        ranks_mine[pl.ds(tl * L, L)] = acc
        ids_mine[pl.ds(tl * L, L)] = idx_t

    pltpu.sync_copy(ids_mine, out_sh.at[ranks_mine])
    plsc.subcore_barrier()

    @pl.when(sid == 0)
    def _():
        pltpu.sync_copy(out_sh.at[pl.ds(0, TOPK)], out_hbm)


def _topk_sc(means):
    mesh = plsc.VectorSubcoreMesh(
        core_axis_name="c", subcore_axis_name="s", num_cores=1)
    f = pl.kernel(
        _topk_body,
        out_type=jax.ShapeDtypeStruct((TOPK,), jnp.int32),
        mesh=mesh,
        scratch_types=[
            pltpu.VMEM((C,), jnp.float32),
            pltpu.VMEM((CPT,), jnp.int32),
            pltpu.VMEM((CPT,), jnp.int32),
            pltpu.VMEM_SHARED((C,), jnp.int32),
            pltpu.VMEM((TOPK,), jnp.int32),
        ],
    )
    return f(means)


@jax.jit
def kernel(input):
    means = _channel_means(input.reshape(C, HW)).reshape(C)
    return _topk_sc(means)


# speedup vs baseline: 1.0081x; 1.0081x over previous
import jax
import jax.numpy as jnp
from jax import lax
from jax.experimental import pallas as pl
from jax.experimental.pallas import tpu as pltpu
from jax.experimental.pallas import tpu_sc as plsc

C = 768
HW = 112 * 112
TOPK = 384
L = 16
NSUB = 16
CPT = C // NSUB
NVREG = C // L

NBUF = 4
CB = 96
NCHUNK = C // CB


def _mean_body(x_hbm, o_ref, *scratch):
    bufs, sems = scratch[:NBUF], scratch[NBUF:]

    def copy(c):
        b = c % NBUF
        return pltpu.make_async_copy(
            x_hbm.at[pl.ds(c * CB, CB)], bufs[b], sems[b])

    for c in range(min(NBUF, NCHUNK)):
        copy(c).start()
    for c in range(NCHUNK):
        copy(c).wait()
        x = bufs[c % NBUF][...]
        o_ref[pl.ds(c * CB, CB)] = (
            jnp.sum(x, axis=1, keepdims=True) * (1.0 / HW))
        if c + NBUF < NCHUNK:
            copy(c + NBUF).start()


def _channel_means(x2):
    return pl.pallas_call(
        _mean_body,
        in_specs=[pl.BlockSpec(memory_space=pltpu.MemorySpace.HBM)],
        out_specs=pl.BlockSpec(memory_space=pltpu.MemorySpace.VMEM),
        out_shape=jax.ShapeDtypeStruct((C, 1), jnp.float32),
        scratch_shapes=(
            [pltpu.VMEM((CB, HW), jnp.float32)] * NBUF
            + [pltpu.SemaphoreType.DMA] * NBUF),
    )(x2)


def _topk_body(means_hbm, out_hbm, means_v, ranks_mine, ids_mine, out_sh):
    sid = lax.axis_index("s")
    iota = jnp.arange(L, dtype=jnp.int32)
    ones = jnp.ones((L,), jnp.int32)
    zeros = jnp.zeros((L,), jnp.int32)

    pltpu.sync_copy(means_hbm, means_v)

    for tl in range(CPT // L):
        tbase = (sid * (CPT // L) + tl) * L
        v = means_v[pl.ds(tbase, L)]
        idx_t = iota + tbase

        @plsc.parallel_loop(0, NVREG, unroll=4, carry=zeros)
        def acc(m, acc, v=v, idx_t=idx_t):
            u = means_v[pl.ds(m * L, L)]
            for k in range(L):
                us = u.at[jnp.full((L,), k, jnp.int32)].get(
                    mode='promise_in_bounds')
                j = m * L + k
                cond = (us > v) | ((us == v) & (j > idx_t))
                acc = acc + jnp.where(cond, ones, zeros)
            return acc
        ranks_mine[pl.ds(tl * L, L)] = acc
        ids_mine[pl.ds(tl * L, L)] = idx_t

    pltpu.sync_copy(ids_mine, out_sh.at[ranks_mine])
    plsc.subcore_barrier()

    @pl.when(sid == 0)
    def _():
        pltpu.sync_copy(out_sh.at[pl.ds(0, TOPK)], out_hbm)


def _topk_sc(means):
    mesh = plsc.VectorSubcoreMesh(
        core_axis_name="c", subcore_axis_name="s", num_cores=1)
    f = pl.kernel(
        _topk_body,
        out_type=jax.ShapeDtypeStruct((TOPK,), jnp.int32),
        mesh=mesh,
        scratch_types=[
            pltpu.VMEM((C,), jnp.float32),
            pltpu.VMEM((CPT,), jnp.int32),
            pltpu.VMEM((CPT,), jnp.int32),
            pltpu.VMEM_SHARED((C,), jnp.int32),
        ],
    )
    return f(means)


@jax.jit
def kernel(input):
    means = _channel_means(input.reshape(C, HW)).reshape(C)
    return _topk_sc(means)
